# Initial kernel scaffold; baseline (speedup 1.0000x reference)
import jax
import jax.numpy as jnp
from jax.experimental import pallas as pl


def kernel(x, CI, rW1, rb1, rW2, rb2, Wexp, Bexp, T1w, T1b, T2w, T2b, Pw, Pb):
    raise NotImplementedError("write your pallas kernel here")


# speedup vs baseline: 5.4118x; 5.4118x over previous
import jax
import jax.numpy as jnp
from jax.experimental import pallas as pl
from jax.experimental.pallas import tpu as pltpu

_B, _L, _N = 4, 2048, 32
_D, _P = 768, 720
_E = 16
_BN = _B * _N


def _moe_kernel(xt_ref, CI_ref, rW1_ref, rb1_ref, rW2_ref, rb2_ref,
                Wexp_ref, Bexp_ref, T1w_ref, T1b_ref, T2w_ref, T2b_ref,
                Pw_ref, Pb_ref, out_ref,
                xn_ref, mean_ref, std_ref, g_ref, acc_ref):
    e = pl.program_id(0)

    @pl.when(e == 0)
    def _prologue():
        xt = xt_ref[...]
        mean = jnp.mean(xt, axis=1, keepdims=True)
        diff = xt - mean
        var = jnp.sum(diff * diff, axis=1, keepdims=True) / (_L - 1)
        std = jnp.sqrt(var) + 1e-6
        mean_ref[...] = mean
        std_ref[...] = std
        xn_ref[...] = diff / std
        h = jnp.maximum(
            jnp.dot(CI_ref[...], rW1_ref[...],
                    preferred_element_type=jnp.float32) + rb1_ref[...], 0.0)
        logits = jnp.dot(h, rW2_ref[...],
                         preferred_element_type=jnp.float32) + rb2_ref[...]
        m = jnp.max(logits, axis=1, keepdims=True)
        p = jnp.exp(logits - m)
        gg = p / jnp.sum(p, axis=1, keepdims=True)
        g_ref[...] = jnp.concatenate([gg, gg, gg, gg], axis=0)
        acc_ref[...] = jnp.zeros_like(acc_ref)

    lane = jax.lax.broadcasted_iota(jnp.int32, (_BN, _E), 1)
    gcol = jnp.sum(jnp.where(lane == e, g_ref[...], 0.0), axis=1,
                   keepdims=True)
    xs = xn_ref[...] * gcol
    acc_ref[...] += jnp.dot(xs, Wexp_ref[0],
                            preferred_element_type=jnp.float32)

    @pl.when(e == _E - 1)
    def _epilogue():
        bexp = Bexp_ref[...]
        emb = acc_ref[...] + jnp.concatenate([bexp] * _B, axis=0)
        t = jnp.maximum(
            jnp.dot(emb, T1w_ref[...],
                    preferred_element_type=jnp.float32) + T1b_ref[...], 0.0)
        x2 = jnp.dot(t, T2w_ref[...],
                     preferred_element_type=jnp.float32) + T2b_ref[...] + emb
        pred = jnp.dot(x2, Pw_ref[...],
                       preferred_element_type=jnp.float32) + Pb_ref[...]
        out_ref[...] = pred * std_ref[...] + mean_ref[...]


def kernel(x, CI, rW1, rb1, rW2, rb2, Wexp, Bexp, T1w, T1b, T2w, T2b, Pw, Pb):
    xt = jnp.transpose(x, (0, 2, 1)).reshape(_BN, _L)
    row = lambda v: v.reshape(1, -1)

    whole = lambda shape: pl.BlockSpec(shape, lambda e: (0,) * len(shape))

    out = pl.pallas_call(
        _moe_kernel,
        grid=(_E,),
        in_specs=[
            whole((_BN, _L)),
            whole((_N, 64)),
            whole((64, 128)),
            whole((1, 128)),
            whole((128, _E)),
            whole((1, _E)),
            pl.BlockSpec((1, _L, _D), lambda e: (e, 0, 0)),
            whole((_N, _D)),
            whole((_D, _D)),
            whole((1, _D)),
            whole((_D, _D)),
            whole((1, _D)),
            whole((_D, _P)),
            whole((1, _P)),
        ],
        out_specs=whole((_BN, _P)),
        out_shape=jax.ShapeDtypeStruct((_BN, _P), jnp.float32),
        scratch_shapes=[
            pltpu.VMEM((_BN, _L), jnp.float32),
            pltpu.VMEM((_BN, 1), jnp.float32),
            pltpu.VMEM((_BN, 1), jnp.float32),
            pltpu.VMEM((_BN, _E), jnp.float32),
            pltpu.VMEM((_BN, _D), jnp.float32),
        ],
        compiler_params=pltpu.CompilerParams(
            dimension_semantics=("arbitrary",),
        ),
    )(xt, CI, rW1, row(rb1), rW2, row(rb2), Wexp, Bexp,
      T1w, row(T1b), T2w, row(T2b), Pw, row(Pb))

    return jnp.transpose(out.reshape(_B, _N, _P), (0, 2, 1))


# speedup vs baseline: 5.4997x; 1.0162x over previous
import jax
import jax.numpy as jnp
from jax.experimental import pallas as pl
from jax.experimental.pallas import tpu as pltpu

_B, _L, _N = 4, 2048, 32
_D, _P = 768, 720
_E = 16
_BN = _B * _N


def _moe_kernel(xt_ref, CI_ref, rW1_ref, rb1_ref, rW2_ref, rb2_ref,
                Wexp_ref, Bexp_ref, T1w_ref, T1b_ref, T2w_ref, T2b_ref,
                Pw_ref, Pb_ref, out_ref,
                xn_ref, mean_ref, std_ref, g_ref, acc_ref):
    e = pl.program_id(0)

    @pl.when(e == 0)
    def _prologue():
        xt = xt_ref[...]
        mean = jnp.mean(xt, axis=1, keepdims=True)
        diff = xt - mean
        var = jnp.sum(diff * diff, axis=1, keepdims=True) / (_L - 1)
        std = jnp.sqrt(var) + 1e-6
        mean_ref[...] = mean
        std_ref[...] = std
        xn_ref[...] = diff / std
        h = jnp.maximum(
            jnp.dot(CI_ref[...], rW1_ref[...],
                    preferred_element_type=jnp.float32) + rb1_ref[...], 0.0)
        logits = jnp.dot(h, rW2_ref[...],
                         preferred_element_type=jnp.float32) + rb2_ref[...]
        m = jnp.max(logits, axis=1, keepdims=True)
        p = jnp.exp(logits - m)
        gg = p / jnp.sum(p, axis=1, keepdims=True)
        g_ref[...] = jnp.concatenate([gg, gg, gg, gg], axis=0)
        acc_ref[...] = jnp.zeros_like(acc_ref)

    lane = jax.lax.broadcasted_iota(jnp.int32, (_BN, _E), 1)
    gcol = jnp.sum(jnp.where(lane == e, g_ref[...], 0.0), axis=1,
                   keepdims=True)
    y = jnp.dot(xn_ref[...], Wexp_ref[0],
                preferred_element_type=jnp.float32)
    acc_ref[...] += gcol * y

    @pl.when(e == _E - 1)
    def _epilogue():
        bexp = Bexp_ref[...]
        emb = acc_ref[...] + jnp.concatenate([bexp] * _B, axis=0)
        t = jnp.maximum(
            jnp.dot(emb, T1w_ref[...],
                    preferred_element_type=jnp.float32) + T1b_ref[...], 0.0)
        x2 = jnp.dot(t, T2w_ref[...],
                     preferred_element_type=jnp.float32) + T2b_ref[...] + emb
        pred = jnp.dot(x2, Pw_ref[...],
                       preferred_element_type=jnp.float32) + Pb_ref[...]
        out_ref[...] = pred * std_ref[...] + mean_ref[...]


def kernel(x, CI, rW1, rb1, rW2, rb2, Wexp, Bexp, T1w, T1b, T2w, T2b, Pw, Pb):
    xt = jnp.transpose(x, (0, 2, 1)).reshape(_BN, _L)
    row = lambda v: v.reshape(1, -1)

    whole = lambda shape: pl.BlockSpec(shape, lambda e: (0,) * len(shape))

    out = pl.pallas_call(
        _moe_kernel,
        grid=(_E,),
        in_specs=[
            whole((_BN, _L)),
            whole((_N, 64)),
            whole((64, 128)),
            whole((1, 128)),
            whole((128, _E)),
            whole((1, _E)),
            pl.BlockSpec((1, _L, _D), lambda e: (e, 0, 0)),
            whole((_N, _D)),
            whole((_D, _D)),
            whole((1, _D)),
            whole((_D, _D)),
            whole((1, _D)),
            whole((_D, _P)),
            whole((1, _P)),
        ],
        out_specs=whole((_BN, _P)),
        out_shape=jax.ShapeDtypeStruct((_BN, _P), jnp.float32),
        scratch_shapes=[
            pltpu.VMEM((_BN, _L), jnp.float32),
            pltpu.VMEM((_BN, 1), jnp.float32),
            pltpu.VMEM((_BN, 1), jnp.float32),
            pltpu.VMEM((_BN, _E), jnp.float32),
            pltpu.VMEM((_BN, _D), jnp.float32),
        ],
        compiler_params=pltpu.CompilerParams(
            dimension_semantics=("arbitrary",),
        ),
    )(xt, CI, rW1, row(rb1), rW2, row(rb2), Wexp, Bexp,
      T1w, row(T1b), T2w, row(T2b), Pw, row(Pb))

    return jnp.transpose(out.reshape(_B, _N, _P), (0, 2, 1))
